# baseline (speedup 1.0000x reference)
import functools

import jax
import jax.numpy as jnp
from jax import lax
from jax.experimental import pallas as pl
from jax.experimental.pallas import tpu as pltpu
from jax.experimental.pallas import tpu_sc as plsc

DIM = 32
NUM_CORES = 2
NUM_SUBCORES = 16
NW = NUM_CORES * NUM_SUBCORES
LN = 16


@functools.lru_cache(maxsize=None)
def _make_lookup(R: int, T: int, V: int):
    bw = R // NW
    bt_w = bw // 128
    dt_n = DIM // 8
    mesh = plsc.VectorSubcoreMesh(core_axis_name="c", subcore_axis_name="s")

    @functools.partial(
        pl.kernel,
        out_type=jax.ShapeDtypeStruct((T, dt_n, R // 128, 8, 128), jnp.float32),
        mesh=mesh,
        compiler_params=pltpu.CompilerParams(
            use_tc_tiling_on_sc=False, needs_layout_passes=False
        ),
        scratch_types=[
            pltpu.VMEM((T, bw), jnp.int32),
            pltpu.VMEM((bw, DIM), jnp.float32),
            pltpu.VMEM((bw, DIM), jnp.float32),
            pltpu.VMEM((dt_n, bt_w, 8, 128), jnp.float32),
            pltpu.VMEM((dt_n, bt_w, 8, 128), jnp.float32),
            pltpu.SemaphoreType.DMA,
            pltpu.SemaphoreType.DMA,
            pltpu.SemaphoreType.DMA,
            pltpu.SemaphoreType.DMA,
        ],
    )
    def lookup(table_hbm, idx_hbm, out_hbm, idx_v, rb0, rb1, ob0, ob1, g0, g1, o0, o1):
        wid = lax.axis_index("s") * NUM_CORES + lax.axis_index("c")
        pltpu.sync_copy(idx_hbm.at[wid], idx_v)

        rbufs = (rb0, rb1)
        obufs = (ob0, ob1)
        gsems = (g0, g1)
        osems = (o0, o1)
        iota = lax.iota(jnp.int32, LN)

        def gather_start(t, slot):
            pltpu.async_copy(table_hbm.at[idx_v.at[t]], rbufs[slot], gsems[slot])

        def wb_copy(t, slot):
            return pltpu.make_async_copy(
                obufs[slot],
                out_hbm.at[t, :, pl.ds(bt_w * wid, bt_w)],
                osems[slot],
            )

        gather_start(0, 0)
        gather_start(1, 1)

        def step(t, slot):
            pltpu.make_async_copy(
                table_hbm.at[idx_v.at[t]], rbufs[slot], gsems[slot]
            ).wait()

            @pl.when(t >= 2)
            def _():
                wb_copy(t - 2, slot).wait()

            rb, ob = rbufs[slot], obufs[slot]

            @plsc.parallel_loop(0, dt_n * bt_w, unroll=4)
            def _(k):
                dt = k // bt_w
                bt = k % bt_w
                rows = [bt * 128 + blk * LN + iota for blk in range(128 // LN)]
                for ds in range(8):
                    col = jnp.full((LN,), dt * 8 + ds, jnp.int32)
                    for blk in range(128 // LN):
                        ob[dt, bt, ds, pl.ds(blk * LN, LN)] = plsc.load_gather(
                            rb, [rows[blk], col]
                        )

            wb_copy(t, slot).start()
            nxt = t + 2

            @pl.when(nxt < T)
            def _():
                gather_start(nxt, slot)

        @pl.loop(0, T, step=2)
        def _(t):
            step(t, 0)
            step(t + 1, 1)

        for slot in range(2):
            wb_copy(T - 2 + slot, slot).wait()

    return lookup


def kernel(X, embed):
    R, T = X.shape
    V, D = embed.shape
    idx = (
        X.T.reshape(T, NW, R // NW).transpose(1, 0, 2).astype(jnp.int32)
    )
    out5 = _make_lookup(R, T, V)(embed, idx)
    return out5.transpose(2, 4, 0, 1, 3).reshape(R, T, D)


# speedup vs baseline: 1.0358x; 1.0358x over previous
import functools

import jax
import jax.numpy as jnp
from jax import lax
from jax.experimental import pallas as pl
from jax.experimental.pallas import tpu as pltpu
from jax.experimental.pallas import tpu_sc as plsc

DIM = 32
NUM_CORES = 2
NUM_SUBCORES = 16
NW = NUM_CORES * NUM_SUBCORES
LN = 16


@functools.lru_cache(maxsize=None)
def _make_lookup(R: int, T: int, V: int):
    bw = R // NW
    bt_w = bw // 128
    dt_n = DIM // 8
    mesh = plsc.VectorSubcoreMesh(core_axis_name="c", subcore_axis_name="s")

    @functools.partial(
        pl.kernel,
        out_type=jax.ShapeDtypeStruct((T, dt_n, R // 128, 8, 128), jnp.float32),
        mesh=mesh,
        compiler_params=pltpu.CompilerParams(
            use_tc_tiling_on_sc=False, needs_layout_passes=False
        ),
        scratch_types=[
            pltpu.VMEM((T, bw), jnp.int32),
            pltpu.VMEM((bw, DIM), jnp.float32),
            pltpu.VMEM((bw, DIM), jnp.float32),
            pltpu.VMEM((dt_n, bt_w, 8, 128), jnp.float32),
            pltpu.VMEM((dt_n, bt_w, 8, 128), jnp.float32),
            pltpu.SemaphoreType.DMA,
            pltpu.SemaphoreType.DMA,
            pltpu.SemaphoreType.DMA,
            pltpu.SemaphoreType.DMA,
        ],
    )
    def lookup(table_hbm, idx_hbm, out_hbm, idx_v, rb0, rb1, ob0, ob1, g0, g1, o0, o1):
        wid = lax.axis_index("s") * NUM_CORES + lax.axis_index("c")
        pltpu.sync_copy(idx_hbm.at[wid], idx_v)

        rbufs = (rb0, rb1)
        obufs = (ob0, ob1)
        gsems = (g0, g1)
        osems = (o0, o1)
        iota = lax.iota(jnp.int32, LN)

        def gather_start(t, slot):
            pltpu.async_copy(table_hbm.at[idx_v.at[t]], rbufs[slot], gsems[slot])

        def wb_copy(t, slot):
            return pltpu.make_async_copy(
                obufs[slot],
                out_hbm.at[t, :, pl.ds(bt_w * wid, bt_w)],
                osems[slot],
            )

        gather_start(0, 0)
        gather_start(1, 1)

        def step(t, slot):
            pltpu.make_async_copy(
                table_hbm.at[idx_v.at[t]], rbufs[slot], gsems[slot]
            ).wait()

            @pl.when(t >= 2)
            def _():
                wb_copy(t - 2, slot).wait()

            rb, ob = rbufs[slot], obufs[slot]

            @plsc.parallel_loop(0, dt_n * bt_w, unroll=2)
            def _(k):
                dt = k // bt_w
                bt = k % bt_w
                rows = [bt * 128 + blk * LN + iota for blk in range(128 // LN)]
                for ds in range(8):
                    col = jnp.full((LN,), dt * 8 + ds, jnp.int32)
                    for blk in range(128 // LN):
                        ob[dt, bt, ds, pl.ds(blk * LN, LN)] = plsc.load_gather(
                            rb, [rows[blk], col]
                        )

            wb_copy(t, slot).start()
            nxt = t + 2

            @pl.when(nxt < T)
            def _():
                gather_start(nxt, slot)

        @pl.loop(0, T, step=2)
        def _(t):
            step(t, 0)
            step(t + 1, 1)

        for slot in range(2):
            wb_copy(T - 2 + slot, slot).wait()

    return lookup


def kernel(X, embed):
    R, T = X.shape
    V, D = embed.shape
    idx = (
        X.T.reshape(T, NW, R // NW).transpose(1, 0, 2).astype(jnp.int32)
    )
    out5 = _make_lookup(R, T, V)(embed, idx)
    return out5.transpose(2, 4, 0, 1, 3).reshape(R, T, D)


# speedup vs baseline: 1.1695x; 1.1291x over previous
import functools

import jax
import jax.numpy as jnp
from jax import lax
from jax.experimental import pallas as pl
from jax.experimental.pallas import tpu as pltpu
from jax.experimental.pallas import tpu_sc as plsc

DIM = 32
NUM_CORES = 2
NUM_SUBCORES = 16
NW = NUM_CORES * NUM_SUBCORES
LN = 16


@functools.lru_cache(maxsize=None)
def _make_lookup(R: int, T: int, V: int):
    bw = R // NW
    bt_w = bw // 128
    dt_n = DIM // 8
    mesh = plsc.VectorSubcoreMesh(core_axis_name="c", subcore_axis_name="s")

    @functools.partial(
        pl.kernel,
        out_type=jax.ShapeDtypeStruct((T, dt_n, R // 128, 8, 128), jnp.float32),
        mesh=mesh,
        compiler_params=pltpu.CompilerParams(
            use_tc_tiling_on_sc=False, needs_layout_passes=False
        ),
        scratch_types=[
            pltpu.VMEM((T, bw), jnp.int32),
            pltpu.VMEM((bw, DIM), jnp.float32),
            pltpu.VMEM((bw, DIM), jnp.float32),
            pltpu.VMEM((dt_n, bt_w, 8, 128), jnp.float32),
            pltpu.VMEM((dt_n, bt_w, 8, 128), jnp.float32),
            pltpu.SemaphoreType.DMA,
            pltpu.SemaphoreType.DMA,
            pltpu.SemaphoreType.DMA,
            pltpu.SemaphoreType.DMA,
        ],
    )
    def lookup(table_hbm, idx_hbm, out_hbm, idx_v, rb0, rb1, ob0, ob1, g0, g1, o0, o1):
        wid = lax.axis_index("s") * NUM_CORES + lax.axis_index("c")
        pltpu.sync_copy(idx_hbm.at[wid], idx_v)

        rbufs = (rb0, rb1)
        obufs = (ob0, ob1)
        gsems = (g0, g1)
        osems = (o0, o1)
        iota = lax.iota(jnp.int32, LN)

        def gather_start(t, slot):
            pltpu.async_copy(table_hbm.at[idx_v.at[t]], rbufs[slot], gsems[slot])

        def wb_copy(t, slot):
            return pltpu.make_async_copy(
                obufs[slot],
                out_hbm.at[t, :, pl.ds(bt_w * wid, bt_w)],
                osems[slot],
            )

        gather_start(0, 0)
        gather_start(1, 1)

        def step(t, slot):
            pltpu.make_async_copy(
                table_hbm.at[idx_v.at[t]], rbufs[slot], gsems[slot]
            ).wait()

            @pl.when(t >= 2)
            def _():
                wb_copy(t - 2, slot).wait()

            rb, ob = rbufs[slot], obufs[slot]

            @plsc.parallel_loop(0, dt_n * bt_w * 8, unroll=4)
            def _(k):
                dt = k // (bt_w * 8)
                bt = (k // 8) % bt_w
                ds = k % 8
                col = jnp.full((LN,), dt * 8 + ds, jnp.int32)
                for blk in range(128 // LN):
                    row = bt * 128 + blk * LN + iota
                    ob[dt, bt, ds, pl.ds(blk * LN, LN)] = plsc.load_gather(
                        rb, [row, col]
                    )

            wb_copy(t, slot).start()
            nxt = t + 2

            @pl.when(nxt < T)
            def _():
                gather_start(nxt, slot)

        @pl.loop(0, T, step=2)
        def _(t):
            step(t, 0)
            step(t + 1, 1)

        for slot in range(2):
            wb_copy(T - 2 + slot, slot).wait()

    return lookup


def kernel(X, embed):
    R, T = X.shape
    V, D = embed.shape
    idx = (
        X.T.reshape(T, NW, R // NW).transpose(1, 0, 2).astype(jnp.int32)
    )
    out5 = _make_lookup(R, T, V)(embed, idx)
    return out5.transpose(2, 4, 0, 1, 3).reshape(R, T, D)


# speedup vs baseline: 1.1835x; 1.0120x over previous
import functools

import jax
import jax.numpy as jnp
from jax import lax
from jax.experimental import pallas as pl
from jax.experimental.pallas import tpu as pltpu
from jax.experimental.pallas import tpu_sc as plsc

DIM = 32
NUM_CORES = 2
NUM_SUBCORES = 16
NW = NUM_CORES * NUM_SUBCORES
LN = 16


@functools.lru_cache(maxsize=None)
def _make_lookup(R: int, T: int, V: int):
    bw = R // NW
    bt_w = bw // 128
    dt_n = DIM // 8
    mesh = plsc.VectorSubcoreMesh(core_axis_name="c", subcore_axis_name="s")

    @functools.partial(
        pl.kernel,
        out_type=jax.ShapeDtypeStruct((T, dt_n, R // 128, 8, 128), jnp.float32),
        mesh=mesh,
        compiler_params=pltpu.CompilerParams(
            use_tc_tiling_on_sc=False, needs_layout_passes=False
        ),
        scratch_types=[
            pltpu.VMEM((T, bw), jnp.int32),
            pltpu.VMEM((bw, DIM), jnp.float32),
            pltpu.VMEM((bw, DIM), jnp.float32),
            pltpu.VMEM((dt_n, bt_w, 8, 128), jnp.float32),
            pltpu.VMEM((dt_n, bt_w, 8, 128), jnp.float32),
            pltpu.SemaphoreType.DMA,
            pltpu.SemaphoreType.DMA,
            pltpu.SemaphoreType.DMA,
            pltpu.SemaphoreType.DMA,
        ],
    )
    def lookup(table_hbm, idx_hbm, out_hbm, idx_v, rb0, rb1, ob0, ob1, g0, g1, o0, o1):
        wid = lax.axis_index("s") * NUM_CORES + lax.axis_index("c")
        pltpu.sync_copy(idx_hbm.at[wid], idx_v)

        rbufs = (rb0, rb1)
        obufs = (ob0, ob1)
        gsems = (g0, g1)
        osems = (o0, o1)
        iota = lax.iota(jnp.int32, LN)

        def gather_start(t, slot):
            pltpu.async_copy(table_hbm.at[idx_v.at[t]], rbufs[slot], gsems[slot])

        def wb_copy(t, slot):
            return pltpu.make_async_copy(
                obufs[slot],
                out_hbm.at[t, :, pl.ds(bt_w * wid, bt_w)],
                osems[slot],
            )

        gather_start(0, 0)
        gather_start(1, 1)

        def step(t, slot):
            pltpu.make_async_copy(
                table_hbm.at[idx_v.at[t]], rbufs[slot], gsems[slot]
            ).wait()

            @pl.when(t >= 2)
            def _():
                wb_copy(t - 2, slot).wait()

            rb, ob = rbufs[slot], obufs[slot]

            @plsc.parallel_loop(0, dt_n * bt_w * 8, unroll=8)
            def _(k):
                dt = k // (bt_w * 8)
                bt = (k // 8) % bt_w
                ds = k % 8
                col = jnp.full((LN,), dt * 8 + ds, jnp.int32)
                for blk in range(128 // LN):
                    row = bt * 128 + blk * LN + iota
                    ob[dt, bt, ds, pl.ds(blk * LN, LN)] = plsc.load_gather(
                        rb, [row, col]
                    )

            wb_copy(t, slot).start()
            nxt = t + 2

            @pl.when(nxt < T)
            def _():
                gather_start(nxt, slot)

        @pl.loop(0, T, step=2)
        def _(t):
            step(t, 0)
            step(t + 1, 1)

        for slot in range(2):
            wb_copy(T - 2 + slot, slot).wait()

    return lookup


def kernel(X, embed):
    R, T = X.shape
    V, D = embed.shape
    idx = (
        X.T.reshape(T, NW, R // NW).transpose(1, 0, 2).astype(jnp.int32)
    )
    out5 = _make_lookup(R, T, V)(embed, idx)
    return out5.transpose(2, 4, 0, 1, 3).reshape(R, T, D)
